# Initial kernel scaffold; baseline (speedup 1.0000x reference)
import jax
import jax.numpy as jnp
from jax.experimental import pallas as pl


def kernel(feature_map):
    raise NotImplementedError("write your pallas kernel here")


# speedup vs baseline: 8.1866x; 8.1866x over previous
import functools

import numpy as np
import jax
import jax.numpy as jnp
from jax import lax
from jax.experimental import pallas as pl
from jax.experimental.pallas import tpu as pltpu
from jax.experimental.pallas import tpu_sc as plsc

_C, _D, _H, _W = 96, 64, 96, 96
_P = 1024
_NB = 8
_NW = 32
_CD = _C * 8
_CD_PER_W = _CD // _NW
_GV = 8 * 8 * 128 // 16


def _static_tables():
    ds = np.arange(0, _D - 8 + 1, 8)
    hs = np.arange(0, _H - 8 + 1, 8)
    ws = np.arange(0, _W - 8 + 1, 8)
    coords = np.stack(np.meshgrid(ds, hs, ws, indexing="ij"), axis=-1).reshape(-1, 3)
    n_all = coords.shape[0]
    idx = np.linspace(0.0, float(n_all - 1), _P).astype(np.int64)
    sel = coords[idx]
    h0 = sel[:, 1].reshape(_NB, 128)
    w0 = sel[:, 2].reshape(_NB, 128)
    hh = np.arange(8)[:, None, None]
    ww = np.arange(8)[None, :, None]
    tab = (h0[:, None, None, :] + hh) * 128 + (w0[:, None, None, :] + ww)
    return tab.reshape(_NB, 8192).astype(np.int32), sel.astype(np.float32)


_SC_MESH = plsc.VectorSubcoreMesh(core_axis_name="c", subcore_axis_name="s")


@functools.partial(
    pl.kernel,
    mesh=_SC_MESH,
    out_type=jax.ShapeDtypeStruct((_C, 8, 8, _NB, 8, 128), jnp.float32),
    scratch_types=[
        pltpu.VMEM((8192,), jnp.int32),
        pltpu.VMEM((_H, _W), jnp.float32),
        pltpu.VMEM((8, 8, 128), jnp.float32),
    ],
    compiler_params=pltpu.CompilerParams(needs_layout_passes=False),
)
def _patch_gather(fm_hbm, tab_hbm, out_hbm, idx_v, plane_v, out_v):
    wid = lax.axis_index("s") * 2 + lax.axis_index("c")

    for k in range(_NB):
        pltpu.sync_copy(tab_hbm.at[k], idx_v)

        def task(t, _, k=k):
            cd = wid * _CD_PER_W + t
            c = cd // 8
            dd = cd - c * 8
            pltpu.sync_copy(fm_hbm.at[c, 8 * k + dd], plane_v)

            def gather(j, _):
                packed = idx_v[pl.ds(j * 16, 16)]
                h16 = packed >> 7
                w16 = packed & 127
                vals = plsc.load_gather(plane_v, [h16, w16])
                hh = j // 64
                ww = (j // 8) - hh * 8
                nlb = j - (j // 8) * 8
                out_v[hh, ww, pl.ds(nlb * 16, 16)] = vals
                return 0

            lax.fori_loop(0, _GV, gather, 0, unroll=8)

            for hh in range(8):
                pltpu.sync_copy(out_v.at[hh], out_hbm.at[c, dd, hh, k])
            return 0

        lax.fori_loop(0, _CD_PER_W, task, 0)


def kernel(feature_map):
    tab_np, sel_np = _static_tables()
    tab = jnp.asarray(tab_np)
    out6 = _patch_gather(feature_map, tab)
    patches = jnp.transpose(out6, (3, 5, 0, 1, 2, 4)).reshape(_P, _C, 8, 8, 8)
    coordinates = jnp.asarray(sel_np)
    return patches, coordinates


# speedup vs baseline: 11.1683x; 1.3642x over previous
import functools

import numpy as np
import jax
import jax.numpy as jnp
from jax import lax
from jax.experimental import pallas as pl
from jax.experimental.pallas import tpu as pltpu
from jax.experimental.pallas import tpu_sc as plsc

_C, _D, _H, _W = 96, 64, 96, 96
_P = 1024
_NB = 8
_NW = 32
_CD = _C * 8
_CD_PER_W = _CD // _NW
_GV = 8 * 8 * 128 // 16


def _static_tables():
    ds = np.arange(0, _D - 8 + 1, 8)
    hs = np.arange(0, _H - 8 + 1, 8)
    ws = np.arange(0, _W - 8 + 1, 8)
    coords = np.stack(np.meshgrid(ds, hs, ws, indexing="ij"), axis=-1).reshape(-1, 3)
    n_all = coords.shape[0]
    idx = np.linspace(0.0, float(n_all - 1), _P).astype(np.int64)
    sel = coords[idx]
    h0 = sel[:, 1].reshape(_NB, 128)
    w0 = sel[:, 2].reshape(_NB, 128)
    hh = np.arange(8)[:, None, None]
    ww = np.arange(8)[None, :, None]
    tab = (h0[:, None, None, :] + hh) * 128 + (w0[:, None, None, :] + ww)
    return tab.reshape(_NB, 8192).astype(np.int32), sel.astype(np.float32)


_SC_MESH = plsc.VectorSubcoreMesh(core_axis_name="c", subcore_axis_name="s")


@functools.partial(
    pl.kernel,
    mesh=_SC_MESH,
    out_type=jax.ShapeDtypeStruct((_C, 8, 8, _NB, 8, 128), jnp.float32),
    scratch_types=[
        pltpu.VMEM((8192,), jnp.int32),
        pltpu.VMEM((2, _H, _W), jnp.float32),
        pltpu.VMEM((2, 8, 8, 128), jnp.float32),
        pltpu.SemaphoreType.DMA((2,)),
        pltpu.SemaphoreType.DMA((2,)),
    ],
    compiler_params=pltpu.CompilerParams(needs_layout_passes=False),
)
def _patch_gather(fm_hbm, tab_hbm, out_hbm, idx_v, plane_v, out_v, sem_in, sem_out):
    wid = lax.axis_index("s") * 2 + lax.axis_index("c")

    def _cdd(t):
        cd = wid * _CD_PER_W + t
        c = cd // 8
        return c, cd - c * 8

    def in_copy(k, t, buf):
        c, dd = _cdd(t)
        return pltpu.make_async_copy(
            fm_hbm.at[c, 8 * k + dd], plane_v.at[buf], sem_in.at[buf]
        )

    def out_copy(k, t, buf):
        c, dd = _cdd(t)
        return pltpu.make_async_copy(
            out_v.at[buf], out_hbm.at[c, dd, :, k], sem_out.at[buf]
        )

    for k in range(_NB):
        pltpu.sync_copy(tab_hbm.at[k], idx_v)
        in_copy(k, 0, 0).start()

        def task(t, _, k=k):
            cur = t % 2

            @pl.when(t + 1 < _CD_PER_W)
            def _():
                in_copy(k, t + 1, 1 - cur).start()

            in_copy(k, t, cur).wait()

            @pl.when(t >= 2)
            def _():
                out_copy(k, t - 2, cur).wait()

            def gather(j, _):
                packed = idx_v[pl.ds(j * 16, 16)]
                h16 = packed >> 7
                w16 = packed & 127
                vals = plsc.load_gather(plane_v.at[cur], [h16, w16])
                hh = j // 64
                ww = (j // 8) - hh * 8
                nlb = j - (j // 8) * 8
                out_v[cur, hh, ww, pl.ds(nlb * 16, 16)] = vals
                return 0

            lax.fori_loop(0, _GV, gather, 0, unroll=8)
            out_copy(k, t, cur).start()
            return 0

        lax.fori_loop(0, _CD_PER_W, task, 0)
        out_copy(k, _CD_PER_W - 2, 0).wait()
        out_copy(k, _CD_PER_W - 1, 1).wait()


def kernel(feature_map):
    tab_np, sel_np = _static_tables()
    tab = jnp.asarray(tab_np)
    out6 = _patch_gather(feature_map, tab)
    patches = jnp.transpose(out6, (3, 5, 0, 1, 2, 4)).reshape(_P, _C, 8, 8, 8)
    coordinates = jnp.asarray(sel_np)
    return patches, coordinates


# speedup vs baseline: 32.3117x; 2.8932x over previous
import functools

import numpy as np
import jax
import jax.numpy as jnp
from jax import lax
from jax.experimental import pallas as pl
from jax.experimental.pallas import tpu as pltpu
from jax.experimental.pallas import tpu_sc as plsc

_C, _D, _H, _W = 96, 64, 96, 96
_P = 1024
_NB = 8
_NW = 32
_CD = _C * 8
_CD_PER_W = _CD // _NW
_GV = 8 * 8 * 128 // 16


def _static_tables():
    ds = np.arange(0, _D - 8 + 1, 8)
    hs = np.arange(0, _H - 8 + 1, 8)
    ws = np.arange(0, _W - 8 + 1, 8)
    coords = np.stack(np.meshgrid(ds, hs, ws, indexing="ij"), axis=-1).reshape(-1, 3)
    n_all = coords.shape[0]
    idx = np.linspace(0.0, float(n_all - 1), _P).astype(np.int64)
    sel = coords[idx]
    h0 = sel[:, 1].reshape(_NB, 128)
    w0 = sel[:, 2].reshape(_NB, 128)
    hh = np.arange(8)[:, None, None]
    ww = np.arange(8)[None, :, None]
    tab = (h0[:, None, None, :] + hh) * 128 + (w0[:, None, None, :] + ww)
    return tab.reshape(_NB, 8192).astype(np.int32), sel.astype(np.float32)


_SC_MESH = plsc.VectorSubcoreMesh(core_axis_name="c", subcore_axis_name="s")


@functools.partial(
    pl.kernel,
    mesh=_SC_MESH,
    out_type=jax.ShapeDtypeStruct((_C, 8, 8, _NB, 8, 128), jnp.float32),
    scratch_types=[
        pltpu.VMEM((8192,), jnp.int32),
        pltpu.VMEM((2, _H, _W), jnp.float32),
        pltpu.VMEM((2, 8, 8, 128), jnp.float32),
        pltpu.SemaphoreType.DMA((2,)),
        pltpu.SemaphoreType.DMA((2,)),
    ],
    compiler_params=pltpu.CompilerParams(needs_layout_passes=False),
)
def _patch_gather(fm_hbm, tab_hbm, out_hbm, idx_v, plane_v, out_v, sem_in, sem_out):
    wid = lax.axis_index("s") * 2 + lax.axis_index("c")

    def _cdd(t):
        cd = wid * _CD_PER_W + t
        c = cd // 8
        return c, cd - c * 8

    def in_copy(k, t, buf):
        c, dd = _cdd(t)
        return pltpu.make_async_copy(
            fm_hbm.at[c, 8 * k + dd], plane_v.at[buf], sem_in.at[buf]
        )

    def out_copy(k, t, buf):
        c, dd = _cdd(t)
        return pltpu.make_async_copy(
            out_v.at[buf], out_hbm.at[c, dd, :, k], sem_out.at[buf]
        )

    for k in range(_NB):
        pltpu.sync_copy(tab_hbm.at[k], idx_v)
        in_copy(k, 0, 0).start()

        def task(t, _, k=k):
            cur = t % 2

            @pl.when(t + 1 < _CD_PER_W)
            def _():
                in_copy(k, t + 1, 1 - cur).start()

            in_copy(k, t, cur).wait()

            @pl.when(t >= 2)
            def _():
                out_copy(k, t - 2, cur).wait()

            @plsc.parallel_loop(0, _GV, unroll=8)
            def gather(j):
                packed = idx_v[pl.ds(j * 16, 16)]
                h16 = packed >> 7
                w16 = packed & 127
                vals = plsc.load_gather(plane_v.at[cur], [h16, w16])
                hh = j // 64
                ww = (j // 8) - hh * 8
                nlb = j - (j // 8) * 8
                out_v[cur, hh, ww, pl.ds(nlb * 16, 16)] = vals
            out_copy(k, t, cur).start()
            return 0

        lax.fori_loop(0, _CD_PER_W, task, 0)
        out_copy(k, _CD_PER_W - 2, 0).wait()
        out_copy(k, _CD_PER_W - 1, 1).wait()


def kernel(feature_map):
    tab_np, sel_np = _static_tables()
    tab = jnp.asarray(tab_np)
    out6 = _patch_gather(feature_map, tab)
    patches = jnp.transpose(out6, (3, 5, 0, 1, 2, 4)).reshape(_P, _C, 8, 8, 8)
    coordinates = jnp.asarray(sel_np)
    return patches, coordinates


# speedup vs baseline: 35.1007x; 1.0863x over previous
import functools

import numpy as np
import jax
import jax.numpy as jnp
from jax import lax
from jax.experimental import pallas as pl
from jax.experimental.pallas import tpu as pltpu
from jax.experimental.pallas import tpu_sc as plsc

_C, _D, _H, _W = 96, 64, 96, 96
_P = 1024
_NB = 8
_NW = 32
_CD = _C * 8
_CD_PER_W = _CD // _NW
_GV = 8 * 8 * 128 // 16


def _static_tables():
    ds = np.arange(0, _D - 8 + 1, 8)
    hs = np.arange(0, _H - 8 + 1, 8)
    ws = np.arange(0, _W - 8 + 1, 8)
    coords = np.stack(np.meshgrid(ds, hs, ws, indexing="ij"), axis=-1).reshape(-1, 3)
    n_all = coords.shape[0]
    idx = np.linspace(0.0, float(n_all - 1), _P).astype(np.int64)
    sel = coords[idx]
    h0 = sel[:, 1].reshape(_NB, 128)
    w0 = sel[:, 2].reshape(_NB, 128)
    hh = np.arange(8)[:, None, None]
    ww = np.arange(8)[None, :, None]
    tab = (h0[:, None, None, :] + hh) * 128 + (w0[:, None, None, :] + ww)
    return tab.reshape(_NB, 8192).astype(np.int32), sel.astype(np.float32)


_SC_MESH = plsc.VectorSubcoreMesh(core_axis_name="c", subcore_axis_name="s")


@functools.partial(
    pl.kernel,
    mesh=_SC_MESH,
    out_type=jax.ShapeDtypeStruct((_C, 8, 8, _NB, 8, 128), jnp.float32),
    scratch_types=[
        pltpu.VMEM((8192,), jnp.int32),
        pltpu.VMEM((2, _H, _W), jnp.float32),
        pltpu.VMEM((2, 8, 8, 128), jnp.float32),
        pltpu.SemaphoreType.DMA((2,)),
        pltpu.SemaphoreType.DMA((2,)),
    ],
    compiler_params=pltpu.CompilerParams(needs_layout_passes=False),
)
def _patch_gather(fm_hbm, tab_hbm, out_hbm, idx_v, plane_v, out_v, sem_in, sem_out):
    wid = lax.axis_index("s") * 2 + lax.axis_index("c")

    def _cdd(t):
        cd = wid * _CD_PER_W + t
        c = cd // 8
        return c, cd - c * 8

    def in_copy(k, t, buf):
        c, dd = _cdd(t)
        return pltpu.make_async_copy(
            fm_hbm.at[c, 8 * k + dd], plane_v.at[buf], sem_in.at[buf]
        )

    def out_copy(k, t, buf):
        c, dd = _cdd(t)
        return pltpu.make_async_copy(
            out_v.at[buf], out_hbm.at[c, dd, :, k], sem_out.at[buf]
        )

    for k in range(_NB):
        pltpu.sync_copy(tab_hbm.at[k], idx_v)
        in_copy(k, 0, 0).start()

        def task(t, _, k=k):
            cur = t % 2

            @pl.when(t + 1 < _CD_PER_W)
            def _():
                in_copy(k, t + 1, 1 - cur).start()

            in_copy(k, t, cur).wait()

            @pl.when(t >= 2)
            def _():
                out_copy(k, t - 2, cur).wait()

            @plsc.parallel_loop(0, _GV, unroll=16)
            def gather(j):
                packed = idx_v[pl.ds(j * 16, 16)]
                h16 = packed >> 7
                w16 = packed & 127
                vals = plsc.load_gather(plane_v.at[cur], [h16, w16])
                hh = j // 64
                ww = (j // 8) - hh * 8
                nlb = j - (j // 8) * 8
                out_v[cur, hh, ww, pl.ds(nlb * 16, 16)] = vals
            out_copy(k, t, cur).start()
            return 0

        lax.fori_loop(0, _CD_PER_W, task, 0)
        out_copy(k, _CD_PER_W - 2, 0).wait()
        out_copy(k, _CD_PER_W - 1, 1).wait()


def kernel(feature_map):
    tab_np, sel_np = _static_tables()
    tab = jnp.asarray(tab_np)
    out6 = _patch_gather(feature_map, tab)
    patches = jnp.transpose(out6, (3, 5, 0, 1, 2, 4)).reshape(_P, _C, 8, 8, 8)
    coordinates = jnp.asarray(sel_np)
    return patches, coordinates


# speedup vs baseline: 48.4921x; 1.3815x over previous
import functools

import numpy as np
import jax
import jax.numpy as jnp
from jax import lax
from jax.experimental import pallas as pl
from jax.experimental.pallas import tpu as pltpu
from jax.experimental.pallas import tpu_sc as plsc

_C, _D, _H, _W = 96, 64, 96, 96
_P = 1024
_NB = 8
_NW = 32
_CD = _C * 8
_CD_PER_W = _CD // _NW
_GV = 8 * 8 * 128 // 16


def _static_tables():
    ds = np.arange(0, _D - 8 + 1, 8)
    hs = np.arange(0, _H - 8 + 1, 8)
    ws = np.arange(0, _W - 8 + 1, 8)
    coords = np.stack(np.meshgrid(ds, hs, ws, indexing="ij"), axis=-1).reshape(-1, 3)
    n_all = coords.shape[0]
    idx = np.linspace(0.0, float(n_all - 1), _P).astype(np.int64)
    sel = coords[idx]
    h0 = sel[:, 1].reshape(_NB, 128)
    w0 = sel[:, 2].reshape(_NB, 128)
    tab = h0 * 128 + w0
    return tab.astype(np.int32), sel.astype(np.float32)


_SC_MESH = plsc.VectorSubcoreMesh(core_axis_name="c", subcore_axis_name="s")


@functools.partial(
    pl.kernel,
    mesh=_SC_MESH,
    out_type=jax.ShapeDtypeStruct((_C, 8, 8, _NB, 8, 128), jnp.float32),
    scratch_types=[
        pltpu.VMEM((128,), jnp.int32),
        pltpu.VMEM((2, _H, _W), jnp.float32),
        pltpu.VMEM((2, 8, 8, 128), jnp.float32),
        pltpu.SemaphoreType.DMA((2,)),
        pltpu.SemaphoreType.DMA((2,)),
    ],
    compiler_params=pltpu.CompilerParams(needs_layout_passes=False),
)
def _patch_gather(fm_hbm, tab_hbm, out_hbm, idx_v, plane_v, out_v, sem_in, sem_out):
    wid = lax.axis_index("s") * 2 + lax.axis_index("c")

    def _cdd(t):
        cd = wid * _CD_PER_W + t
        c = cd // 8
        return c, cd - c * 8

    def in_copy(k, t, buf):
        c, dd = _cdd(t)
        return pltpu.make_async_copy(
            fm_hbm.at[c, 8 * k + dd], plane_v.at[buf], sem_in.at[buf]
        )

    def out_copy(k, t, buf):
        c, dd = _cdd(t)
        return pltpu.make_async_copy(
            out_v.at[buf], out_hbm.at[c, dd, :, k], sem_out.at[buf]
        )

    for k in range(_NB):
        pltpu.sync_copy(tab_hbm.at[k], idx_v)
        in_copy(k, 0, 0).start()

        def task(t, _, k=k):
            cur = t % 2

            @pl.when(t + 1 < _CD_PER_W)
            def _():
                in_copy(k, t + 1, 1 - cur).start()

            in_copy(k, t, cur).wait()

            @pl.when(t >= 2)
            def _():
                out_copy(k, t - 2, cur).wait()

            bases = [idx_v[pl.ds(i * 16, 16)] for i in range(8)]

            @plsc.parallel_loop(0, 64, unroll=4)
            def gather(m):
                hh = m // 8
                ww = m - hh * 8
                off = hh * 128 + ww
                for nlb in range(8):
                    flat = bases[nlb] + off
                    h16 = flat >> 7
                    w16 = flat & 127
                    vals = plsc.load_gather(plane_v.at[cur], [h16, w16])
                    out_v[cur, hh, ww, pl.ds(nlb * 16, 16)] = vals
            out_copy(k, t, cur).start()
            return 0

        lax.fori_loop(0, _CD_PER_W, task, 0)
        out_copy(k, _CD_PER_W - 2, 0).wait()
        out_copy(k, _CD_PER_W - 1, 1).wait()


def kernel(feature_map):
    tab_np, sel_np = _static_tables()
    tab = jnp.asarray(tab_np)
    out6 = _patch_gather(feature_map, tab)
    patches = jnp.transpose(out6, (3, 5, 0, 1, 2, 4)).reshape(_P, _C, 8, 8, 8)
    coordinates = jnp.asarray(sel_np)
    return patches, coordinates


# speedup vs baseline: 54.1819x; 1.1173x over previous
import functools

import numpy as np
import jax
import jax.numpy as jnp
from jax import lax
from jax.experimental import pallas as pl
from jax.experimental.pallas import tpu as pltpu
from jax.experimental.pallas import tpu_sc as plsc

_C, _D, _H, _W = 96, 64, 96, 96
_P = 1024
_NB = 8
_NW = 32
_CD = _C * 8
_CD_PER_W = _CD // _NW
_GV = 8 * 8 * 128 // 16


def _static_tables():
    ds = np.arange(0, _D - 8 + 1, 8)
    hs = np.arange(0, _H - 8 + 1, 8)
    ws = np.arange(0, _W - 8 + 1, 8)
    coords = np.stack(np.meshgrid(ds, hs, ws, indexing="ij"), axis=-1).reshape(-1, 3)
    n_all = coords.shape[0]
    idx = np.linspace(0.0, float(n_all - 1), _P).astype(np.int64)
    sel = coords[idx]
    h0 = sel[:, 1].reshape(_NB, 128)
    w0 = sel[:, 2].reshape(_NB, 128)
    tab = h0 * 128 + w0
    return tab.astype(np.int32), sel.astype(np.float32)


_SC_MESH = plsc.VectorSubcoreMesh(core_axis_name="c", subcore_axis_name="s")


@functools.partial(
    pl.kernel,
    mesh=_SC_MESH,
    out_type=jax.ShapeDtypeStruct((_C, 8, 8, _NB, 8, 128), jnp.float32),
    scratch_types=[
        pltpu.VMEM((128,), jnp.int32),
        pltpu.VMEM((3, _H, _W), jnp.float32),
        pltpu.VMEM((3, 8, 8, 128), jnp.float32),
        pltpu.SemaphoreType.DMA((3,)),
        pltpu.SemaphoreType.DMA((3,)),
    ],
    compiler_params=pltpu.CompilerParams(needs_layout_passes=False),
)
def _patch_gather(fm_hbm, tab_hbm, out_hbm, idx_v, plane_v, out_v, sem_in, sem_out):
    wid = lax.axis_index("s") * 2 + lax.axis_index("c")

    def _cdd(t):
        cd = wid * _CD_PER_W + t
        c = cd // 8
        return c, cd - c * 8

    def in_copy(k, t, buf):
        c, dd = _cdd(t)
        return pltpu.make_async_copy(
            fm_hbm.at[c, 8 * k + dd], plane_v.at[buf], sem_in.at[buf]
        )

    def out_copy(k, t, buf):
        c, dd = _cdd(t)
        return pltpu.make_async_copy(
            out_v.at[buf], out_hbm.at[c, dd, :, k], sem_out.at[buf]
        )

    for k in range(_NB):
        pltpu.sync_copy(tab_hbm.at[k], idx_v)
        in_copy(k, 0, 0).start()
        in_copy(k, 1, 1).start()

        def task(t, _, k=k):
            cur = t % 3

            @pl.when(t + 2 < _CD_PER_W)
            def _():
                in_copy(k, t + 2, (t + 2) % 3).start()

            in_copy(k, t, cur).wait()

            @pl.when(t >= 3)
            def _():
                out_copy(k, t - 3, cur).wait()

            bases = [idx_v[pl.ds(i * 16, 16)] for i in range(8)]

            @plsc.parallel_loop(0, 64, unroll=4)
            def gather(m):
                hh = m // 8
                ww = m - hh * 8
                off = hh * 128 + ww
                for nlb in range(8):
                    flat = bases[nlb] + off
                    h16 = flat >> 7
                    w16 = flat & 127
                    vals = plsc.load_gather(plane_v.at[cur], [h16, w16])
                    out_v[cur, hh, ww, pl.ds(nlb * 16, 16)] = vals
            out_copy(k, t, cur).start()
            return 0

        lax.fori_loop(0, _CD_PER_W, task, 0)
        for d in (_CD_PER_W - 3, _CD_PER_W - 2, _CD_PER_W - 1):
            out_copy(k, d, d % 3).wait()


def kernel(feature_map):
    tab_np, sel_np = _static_tables()
    tab = jnp.asarray(tab_np)
    out6 = _patch_gather(feature_map, tab)
    patches = jnp.transpose(out6, (3, 5, 0, 1, 2, 4)).reshape(_P, _C, 8, 8, 8)
    coordinates = jnp.asarray(sel_np)
    return patches, coordinates


# speedup vs baseline: 54.6594x; 1.0088x over previous
import functools

import numpy as np
import jax
import jax.numpy as jnp
from jax import lax
from jax.experimental import pallas as pl
from jax.experimental.pallas import tpu as pltpu
from jax.experimental.pallas import tpu_sc as plsc

_C, _D, _H, _W = 96, 64, 96, 96
_P = 1024
_NB = 8
_NW = 32
_CD = _C * 8
_CD_PER_W = _CD // _NW
_GV = 8 * 8 * 128 // 16


def _static_tables():
    ds = np.arange(0, _D - 8 + 1, 8)
    hs = np.arange(0, _H - 8 + 1, 8)
    ws = np.arange(0, _W - 8 + 1, 8)
    coords = np.stack(np.meshgrid(ds, hs, ws, indexing="ij"), axis=-1).reshape(-1, 3)
    n_all = coords.shape[0]
    idx = np.linspace(0.0, float(n_all - 1), _P).astype(np.int64)
    sel = coords[idx]
    h0 = sel[:, 1].reshape(_NB, 128)
    w0 = sel[:, 2].reshape(_NB, 128)
    tab = h0 * 128 + w0
    return tab.astype(np.int32), sel.astype(np.float32)


_SC_MESH = plsc.VectorSubcoreMesh(core_axis_name="c", subcore_axis_name="s")


@functools.partial(
    pl.kernel,
    mesh=_SC_MESH,
    out_type=jax.ShapeDtypeStruct((_C, 8, 8, _NB, 8, 128), jnp.float32),
    scratch_types=[
        pltpu.VMEM((128,), jnp.int32),
        pltpu.VMEM((3, _H, _W), jnp.float32),
        pltpu.VMEM((3, 8, 8, 128), jnp.float32),
        pltpu.SemaphoreType.DMA((3,)),
        pltpu.SemaphoreType.DMA((3,)),
    ],
    compiler_params=pltpu.CompilerParams(needs_layout_passes=False),
)
def _patch_gather(fm_hbm, tab_hbm, out_hbm, idx_v, plane_v, out_v, sem_in, sem_out):
    wid = lax.axis_index("s") * 2 + lax.axis_index("c")

    def _cdd(t):
        cd = wid * _CD_PER_W + t
        c = cd // 8
        return c, cd - c * 8

    def in_copy(k, t, buf):
        c, dd = _cdd(t)
        return pltpu.make_async_copy(
            fm_hbm.at[c, 8 * k + dd], plane_v.at[buf], sem_in.at[buf]
        )

    def out_copy(k, t, buf):
        c, dd = _cdd(t)
        return pltpu.make_async_copy(
            out_v.at[buf], out_hbm.at[c, dd, :, k], sem_out.at[buf]
        )

    for k in range(_NB):
        pltpu.sync_copy(tab_hbm.at[k], idx_v)
        in_copy(k, 0, 0).start()
        in_copy(k, 1, 1).start()

        def task(t, _, k=k):
            cur = t % 3

            @pl.when(t + 2 < _CD_PER_W)
            def _():
                in_copy(k, t + 2, (t + 2) % 3).start()

            in_copy(k, t, cur).wait()

            @pl.when(t >= 3)
            def _():
                out_copy(k, t - 3, cur).wait()

            bases = [idx_v[pl.ds(i * 16, 16)] for i in range(8)]

            @plsc.parallel_loop(0, 64, unroll=8)
            def gather(m):
                hh = m // 8
                ww = m - hh * 8
                off = hh * 128 + ww
                for nlb in range(8):
                    flat = bases[nlb] + off
                    h16 = flat >> 7
                    w16 = flat & 127
                    vals = plsc.load_gather(plane_v.at[cur], [h16, w16])
                    out_v[cur, hh, ww, pl.ds(nlb * 16, 16)] = vals
            out_copy(k, t, cur).start()
            return 0

        lax.fori_loop(0, _CD_PER_W, task, 0)
        for d in (_CD_PER_W - 3, _CD_PER_W - 2, _CD_PER_W - 1):
            out_copy(k, d, d % 3).wait()


def kernel(feature_map):
    tab_np, sel_np = _static_tables()
    tab = jnp.asarray(tab_np)
    out6 = _patch_gather(feature_map, tab)
    patches = jnp.transpose(out6, (3, 5, 0, 1, 2, 4)).reshape(_P, _C, 8, 8, 8)
    coordinates = jnp.asarray(sel_np)
    return patches, coordinates
